# Initial kernel scaffold; baseline (speedup 1.0000x reference)
import jax
import jax.numpy as jnp
from jax.experimental import pallas as pl


def kernel(x, emb, W_ih, W_hh, b_ih, b_hh):
    raise NotImplementedError("write your pallas kernel here")


# speedup vs baseline: 25.7622x; 25.7622x over previous
import functools

import jax
import jax.numpy as jnp
from jax import lax
from jax.experimental import pallas as pl
from jax.experimental.pallas import tpu as pltpu
from jax.experimental.pallas import tpu_sc as plsc

IDX_PER_STREAM = 128
STREAMS_IN_FLIGHT = 8
N_WORKERS = 32
PACK = 4


def _make_gather(n_streams, es):
    cpw = n_streams // N_WORKERS
    sup = cpw // STREAMS_IN_FLIGHT
    rows_per_sup = STREAMS_IN_FLIGHT * IDX_PER_STREAM
    mesh = plsc.VectorSubcoreMesh(core_axis_name="c", subcore_axis_name="s")

    @functools.partial(
        pl.kernel,
        mesh=mesh,
        out_type=jax.ShapeDtypeStruct((n_streams * IDX_PER_STREAM, es),
                                      jnp.float32),
        scratch_types=[
            pltpu.VMEM((cpw, IDX_PER_STREAM), jnp.int32),
            pltpu.VMEM((rows_per_sup, es), jnp.float32),
            pltpu.SemaphoreType.DMA,
        ],
        compiler_params=pltpu.CompilerParams(use_tc_tiling_on_sc=False),
    )
    def gather_k(idx_hbm, table_hbm, out_hbm, idx_v, rows_v, gsem):
        wid = lax.axis_index("s") * 2 + lax.axis_index("c")
        base_stream = wid * cpw
        pltpu.sync_copy(idx_hbm.at[pl.ds(base_stream, cpw)], idx_v)

        def outer(s, carry):
            cps = []
            for j in range(STREAMS_IN_FLIGHT):
                cp = pltpu.async_copy(
                    table_hbm.at[idx_v.at[s * STREAMS_IN_FLIGHT + j]],
                    rows_v.at[pl.ds(j * IDX_PER_STREAM, IDX_PER_STREAM)],
                    gsem,
                )
                cps.append(cp)
            for cp in cps:
                cp.wait()
            row0 = (base_stream + s * STREAMS_IN_FLIGHT) * IDX_PER_STREAM
            pltpu.sync_copy(rows_v, out_hbm.at[pl.ds(row0, rows_per_sup)])
            return carry

        lax.fori_loop(0, sup, outer, 0)

    return gather_k


def _len_body(x_ref, out_ref):
    rows, pw = out_ref.shape
    seq = x_ref.shape[1] // PACK
    parts = []
    for q in range(PACK):
        cnt = jnp.sum(
            (x_ref[:, q * seq : (q + 1) * seq] != 0).astype(jnp.int32),
            axis=1,
            keepdims=True,
        )
        parts.append(jnp.broadcast_to(cnt, (rows, pw // PACK)))
    out_ref[...] = jnp.concatenate(parts, axis=1)


def _gru_body(len_ref, e_ref, wih_ref, whh_ref, b_ref, bhn_ref, out_ref,
              h_scr):
    t = pl.program_id(0)
    n_steps = pl.num_programs(0)
    pw = h_scr.shape[1]

    @pl.when(t == 0)
    def _init():
        h_scr[...] = jnp.zeros_like(h_scr)

    h4 = h_scr[...]
    e_t = e_ref[0]
    ge = jnp.dot(e_t.astype(jnp.bfloat16), wih_ref[...],
                 preferred_element_type=jnp.float32)
    gh = jnp.dot(h4.astype(jnp.bfloat16), whh_ref[...],
                 preferred_element_type=jnp.float32)
    g = ge + b_ref[...]
    rz = jax.nn.sigmoid(g[:, : 2 * pw] + gh[:, : 2 * pw])
    r = rz[:, :pw]
    z = rz[:, pw:]
    n = jnp.tanh(g[:, 2 * pw :] + r * (gh[:, 2 * pw :] + bhn_ref[...]))
    h_new = n + z * (h4 - n)
    keep = t < len_ref[...]
    h_scr[...] = jnp.where(keep, h_new, h4)

    @pl.when(t == n_steps - 1)
    def _fin():
        out_ref[...] = h_scr[...]


def kernel(x, emb, W_ih, W_hh, b_ih, b_hh):
    x = x.astype(jnp.int32)
    bsz, seq = x.shape
    es = emb.shape[1]
    hs = W_hh.shape[1]
    rows = bsz // PACK

    idx = x.T.reshape(seq * bsz)
    n_streams = idx.shape[0] // IDX_PER_STREAM
    idx2d = idx.reshape(n_streams, IDX_PER_STREAM)
    gather = _make_gather(n_streams, es)
    e4 = gather(idx2d, emb).reshape(seq, rows, PACK * es)

    WihT = W_ih.T
    WhhT = W_hh.T
    eye = jnp.eye(PACK, dtype=jnp.float32)

    def gate_major(w):
        return jnp.concatenate(
            [jnp.kron(eye, w[:, i * hs : (i + 1) * hs]) for i in range(3)],
            axis=1,
        )

    WihBD = gate_major(WihT).astype(jnp.bfloat16)
    WhhBD = gate_major(WhhT).astype(jnp.bfloat16)
    b4 = jnp.concatenate(
        [jnp.tile(b_ih[i * hs : (i + 1) * hs]
                  + (b_hh[i * hs : (i + 1) * hs] if i < 2 else 0.0), PACK)
         for i in range(3)]
    ).reshape(1, 3 * PACK * hs)
    bhn = jnp.tile(b_hh[2 * hs :], PACK).reshape(1, PACK * hs)

    x4 = x.reshape(rows, PACK * seq)

    len4 = pl.pallas_call(
        _len_body,
        in_specs=[pl.BlockSpec((rows, PACK * seq), lambda: (0, 0))],
        out_specs=pl.BlockSpec((rows, PACK * hs), lambda: (0, 0)),
        out_shape=jax.ShapeDtypeStruct((rows, PACK * hs), jnp.int32),
    )(x4)

    h4 = pl.pallas_call(
        _gru_body,
        grid=(seq,),
        in_specs=[
            pl.BlockSpec((rows, PACK * hs), lambda t: (0, 0)),
            pl.BlockSpec((1, rows, PACK * es), lambda t: (t, 0, 0)),
            pl.BlockSpec((PACK * es, 3 * PACK * hs), lambda t: (0, 0)),
            pl.BlockSpec((PACK * hs, 3 * PACK * hs), lambda t: (0, 0)),
            pl.BlockSpec((1, 3 * PACK * hs), lambda t: (0, 0)),
            pl.BlockSpec((1, PACK * hs), lambda t: (0, 0)),
        ],
        out_specs=pl.BlockSpec((rows, PACK * hs), lambda t: (0, 0)),
        out_shape=jax.ShapeDtypeStruct((rows, PACK * hs), jnp.float32),
        scratch_shapes=[
            pltpu.VMEM((rows, PACK * hs), jnp.float32),
        ],
        compiler_params=pltpu.CompilerParams(
            dimension_semantics=("arbitrary",)
        ),
    )(len4, e4, WihBD, WhhBD, b4, bhn)
    return h4.reshape(rows, PACK, hs).reshape(bsz, hs)
